# baseline (device time: 199631 ns/iter reference)
import jax
import jax.numpy as jnp
from jax import lax
from jax.experimental import pallas as pl
from jax.experimental.pallas import tpu as pltpu

N_DEV = 4


def _matmul(a, b, block_n):
    m, k = a.shape
    _, n = b.shape

    def body(a_ref, b_ref, o_ref):
        o_ref[...] = jnp.dot(
            a_ref[...], b_ref[...], preferred_element_type=jnp.float32
        )

    return pl.pallas_call(
        body,
        grid=(n // block_n,),
        in_specs=[
            pl.BlockSpec((m, k), lambda j: (0, 0)),
            pl.BlockSpec((k, block_n), lambda j: (0, j)),
        ],
        out_specs=pl.BlockSpec((m, block_n), lambda j: (0, j)),
        out_shape=jax.ShapeDtypeStruct((m, n), jnp.float32),
    )(a, b)


def _allreduce_relu(x, collective_id):
    m, n = x.shape

    def body(x_ref, o_ref, comm_ref, send_sems, recv_sems):
        my = lax.axis_index("i")
        left = lax.rem(my + N_DEV - 1, N_DEV)
        right = lax.rem(my + 1, N_DEV)

        barrier = pltpu.get_barrier_semaphore()
        for nbr in (left, right):
            pl.semaphore_signal(
                barrier, inc=1, device_id=(nbr,),
                device_id_type=pl.DeviceIdType.MESH,
            )
        pl.semaphore_wait(barrier, 2)

        comm_ref[0] = x_ref[...]
        o_ref[...] = x_ref[...]
        for h in range(N_DEV - 1):
            rdma = pltpu.make_async_remote_copy(
                src_ref=comm_ref.at[h],
                dst_ref=comm_ref.at[h + 1],
                send_sem=send_sems.at[h],
                recv_sem=recv_sems.at[h],
                device_id=(right,),
                device_id_type=pl.DeviceIdType.MESH,
            )
            rdma.start()
            rdma.wait()
            o_ref[...] += comm_ref[h + 1]
        o_ref[...] = jnp.maximum(o_ref[...], 0.0)

    return pl.pallas_call(
        body,
        out_shape=jax.ShapeDtypeStruct((m, n), jnp.float32),
        in_specs=[pl.BlockSpec(memory_space=pltpu.VMEM)],
        out_specs=pl.BlockSpec(memory_space=pltpu.VMEM),
        scratch_shapes=[
            pltpu.VMEM((N_DEV, m, n), jnp.float32),
            pltpu.SemaphoreType.DMA((N_DEV - 1,)),
            pltpu.SemaphoreType.DMA((N_DEV - 1,)),
        ],
        compiler_params=pltpu.CompilerParams(collective_id=collective_id),
    )(x)


def kernel(x, Win0, Wout0, Win1, Wout1, Win2, Wout2):
    h = x
    for cid, (Win, Wout) in enumerate(
        ((Win0, Wout0), (Win1, Wout1), (Win2, Wout2))
    ):
        partial = _matmul(h, Win, block_n=512)
        act = _allreduce_relu(partial, collective_id=cid)
        h = _matmul(act, Wout, block_n=512)
    return h


# device time: 148511 ns/iter; 1.3442x vs baseline; 1.3442x over previous
import jax
import jax.numpy as jnp
from jax import lax
from jax.experimental import pallas as pl
from jax.experimental.pallas import tpu as pltpu

N_DEV = 4


def _matmul(a, b, block_n):
    m, k = a.shape
    _, n = b.shape

    def body(a_ref, b_ref, o_ref):
        o_ref[...] = jnp.dot(
            a_ref[...].astype(jnp.bfloat16),
            b_ref[...].astype(jnp.bfloat16),
            preferred_element_type=jnp.float32,
        )

    return pl.pallas_call(
        body,
        grid=(n // block_n,),
        in_specs=[
            pl.BlockSpec((m, k), lambda j: (0, 0)),
            pl.BlockSpec((k, block_n), lambda j: (0, j)),
        ],
        out_specs=pl.BlockSpec((m, block_n), lambda j: (0, j)),
        out_shape=jax.ShapeDtypeStruct((m, n), jnp.float32),
    )(a, b)


def _allreduce_relu(x, collective_id):
    m, n = x.shape

    def body(x_ref, o_ref, comm_ref, send_sems, recv_sems):
        my = lax.axis_index("i")
        left = lax.rem(my + N_DEV - 1, N_DEV)
        right = lax.rem(my + 1, N_DEV)

        barrier = pltpu.get_barrier_semaphore()
        for nbr in (left, right):
            pl.semaphore_signal(
                barrier, inc=1, device_id=(nbr,),
                device_id_type=pl.DeviceIdType.MESH,
            )
        pl.semaphore_wait(barrier, 2)

        comm_ref[0] = x_ref[...].astype(jnp.bfloat16)
        o_ref[...] = x_ref[...]
        for h in range(N_DEV - 1):
            rdma = pltpu.make_async_remote_copy(
                src_ref=comm_ref.at[h],
                dst_ref=comm_ref.at[h + 1],
                send_sem=send_sems.at[h],
                recv_sem=recv_sems.at[h],
                device_id=(right,),
                device_id_type=pl.DeviceIdType.MESH,
            )
            rdma.start()
            rdma.wait()
            o_ref[...] += comm_ref[h + 1].astype(jnp.float32)
        o_ref[...] = jnp.maximum(o_ref[...], 0.0)

    return pl.pallas_call(
        body,
        out_shape=jax.ShapeDtypeStruct((m, n), jnp.float32),
        in_specs=[pl.BlockSpec(memory_space=pltpu.VMEM)],
        out_specs=pl.BlockSpec(memory_space=pltpu.VMEM),
        scratch_shapes=[
            pltpu.VMEM((N_DEV, m, n), jnp.bfloat16),
            pltpu.SemaphoreType.DMA((N_DEV - 1,)),
            pltpu.SemaphoreType.DMA((N_DEV - 1,)),
        ],
        compiler_params=pltpu.CompilerParams(collective_id=collective_id),
    )(x)


def kernel(x, Win0, Wout0, Win1, Wout1, Win2, Wout2):
    h = x
    for cid, (Win, Wout) in enumerate(
        ((Win0, Wout0), (Win1, Wout1), (Win2, Wout2))
    ):
        partial = _matmul(h, Win, block_n=512)
        act = _allreduce_relu(partial, collective_id=cid)
        h = _matmul(act, Wout, block_n=512)
    return h


# device time: 75628 ns/iter; 2.6396x vs baseline; 1.9637x over previous
import jax
import jax.numpy as jnp
from jax import lax
from jax.experimental import pallas as pl
from jax.experimental.pallas import tpu as pltpu

N_DEV = 4


def kernel(x, Win0, Wout0, Win1, Wout1, Win2, Wout2):
    m, kdim = x.shape
    ndim = Win0.shape[1]
    odim = Wout0.shape[1]
    Q = ndim // N_DEV
    C = 512
    HC = Q // C
    NCH = ndim // C
    NB = 3

    def body(x_ref, win0, wout0, win1, wout1, win2, wout2, o_ref,
             act_bf, pown, p1send, p1recv, hbf, p2recv, acc,
             winbuf, woutbuf,
             win_sems, wout_sems,
             p1_send_sems, p1_recv_sems, p2_send_sems, p2_recv_sems):
        my = lax.axis_index("i")
        wins = (win0, win1, win2)
        wouts = (wout0, wout1, wout2)

        barrier = pltpu.get_barrier_semaphore()
        for d in range(1, N_DEV):
            pl.semaphore_signal(
                barrier, inc=1,
                device_id=(lax.rem(my + d, N_DEV),),
                device_id_type=pl.DeviceIdType.MESH,
            )
        pl.semaphore_wait(barrier, N_DEV - 1)

        def issue_win(L, t):
            slot = t % NB
            q = lax.rem(my + 1 + (t // HC), N_DEV)
            col = q * Q + (t % HC) * C
            cp = pltpu.make_async_copy(
                wins[L].at[:, pl.ds(col, C)], winbuf.at[slot],
                win_sems.at[slot])
            cp.start()
            return cp

        def issue_wout(L, t):
            slot = t % NB
            q = lax.rem(my + (t // HC), N_DEV)
            row = q * Q + (t % HC) * C
            cp = pltpu.make_async_copy(
                wouts[L].at[pl.ds(row, C), :], woutbuf.at[slot],
                wout_sems.at[slot])
            cp.start()
            return cp

        def p1_rdma(s):
            peer = lax.rem(my + 1 + s, N_DEV)
            return pltpu.make_async_remote_copy(
                src_ref=p1send.at[s], dst_ref=p1recv.at[2 - s],
                send_sem=p1_send_sems.at[s], recv_sem=p1_recv_sems.at[2 - s],
                device_id=(peer,), device_id_type=pl.DeviceIdType.MESH)

        def p2_rdma(s):
            peer = lax.rem(my + 1 + s, N_DEV)
            return pltpu.make_async_remote_copy(
                src_ref=hbf, dst_ref=p2recv.at[2 - s],
                send_sem=p2_send_sems.at[s], recv_sem=p2_recv_sems.at[2 - s],
                device_id=(peer,), device_id_type=pl.DeviceIdType.MESH)

        def p1_wait(slot):
            pltpu.make_async_remote_copy(
                src_ref=p1send.at[0], dst_ref=p1recv.at[slot],
                send_sem=p1_send_sems.at[0], recv_sem=p1_recv_sems.at[slot],
                device_id=(my,), device_id_type=pl.DeviceIdType.MESH,
            ).wait_recv()

        def p2_wait(slot):
            pltpu.make_async_remote_copy(
                src_ref=hbf, dst_ref=p2recv.at[slot],
                send_sem=p2_send_sems.at[0], recv_sem=p2_recv_sems.at[slot],
                device_id=(my,), device_id_type=pl.DeviceIdType.MESH,
            ).wait_recv()

        act_bf[...] = x_ref[...].astype(jnp.bfloat16)

        win_d = {}
        wout_d = {}
        for t in range(NB):
            win_d[(0, t)] = issue_win(0, t)

        for L in range(3):
            p1_sent = []
            for t in range(NCH):
                win_d.pop((L, t)).wait()
                j, half = divmod(t, HC)
                part = jnp.dot(
                    act_bf[...], winbuf[t % NB].astype(jnp.bfloat16),
                    preferred_element_type=jnp.float32)
                if j < 3:
                    p1send[j, :, half * C:(half + 1) * C] = (
                        part.astype(jnp.bfloat16))
                    if half == HC - 1:
                        r = p1_rdma(j)
                        r.start()
                        p1_sent.append(r)
                else:
                    pown[:, half * C:(half + 1) * C] = part
                if t + NB < NCH:
                    win_d[(L, t + NB)] = issue_win(L, t + NB)

            for t in range(NB):
                wout_d[(L, t)] = issue_wout(L, t)

            for s in range(3):
                p1_wait(s)
            h = (pown[...]
                 + p1recv[0].astype(jnp.float32)
                 + p1recv[1].astype(jnp.float32)
                 + p1recv[2].astype(jnp.float32))
            hbf[...] = jnp.maximum(h, 0.0).astype(jnp.bfloat16)
            p2_sent = []
            for s in range(3):
                r = p2_rdma(s)
                r.start()
                p2_sent.append(r)

            for t in range(NCH):
                j, half = divmod(t, HC)
                if half == 0 and j > 0:
                    p2_wait(j - 1)
                hpart = (hbf[:, half * C:(half + 1) * C] if j == 0
                         else p2recv[j - 1, :, half * C:(half + 1) * C])
                wout_d.pop((L, t)).wait()
                term = jnp.dot(
                    hpart, woutbuf[t % NB].astype(jnp.bfloat16),
                    preferred_element_type=jnp.float32)
                if t == 0:
                    acc[...] = term
                else:
                    acc[...] += term
                if t + NB < NCH:
                    wout_d[(L, t + NB)] = issue_wout(L, t + NB)
                if L < 2 and t < NB:
                    win_d[(L + 1, t)] = issue_win(L + 1, t)

            for r in p1_sent:
                r.wait_send()
            for r in p2_sent:
                r.wait_send()

            if L < 2:
                act_bf[...] = acc[...].astype(jnp.bfloat16)
            else:
                o_ref[...] = acc[...]

    return pl.pallas_call(
        body,
        out_shape=jax.ShapeDtypeStruct((m, odim), jnp.float32),
        in_specs=[
            pl.BlockSpec(memory_space=pltpu.VMEM),
            pl.BlockSpec(memory_space=pl.ANY),
            pl.BlockSpec(memory_space=pl.ANY),
            pl.BlockSpec(memory_space=pl.ANY),
            pl.BlockSpec(memory_space=pl.ANY),
            pl.BlockSpec(memory_space=pl.ANY),
            pl.BlockSpec(memory_space=pl.ANY),
        ],
        out_specs=pl.BlockSpec(memory_space=pltpu.VMEM),
        scratch_shapes=[
            pltpu.VMEM((m, kdim), jnp.bfloat16),
            pltpu.VMEM((m, Q), jnp.float32),
            pltpu.VMEM((3, m, Q), jnp.bfloat16),
            pltpu.VMEM((3, m, Q), jnp.bfloat16),
            pltpu.VMEM((m, Q), jnp.bfloat16),
            pltpu.VMEM((3, m, Q), jnp.bfloat16),
            pltpu.VMEM((m, odim), jnp.float32),
            pltpu.VMEM((NB, kdim, C), jnp.float32),
            pltpu.VMEM((NB, C, odim), jnp.float32),
            pltpu.SemaphoreType.DMA((NB,)),
            pltpu.SemaphoreType.DMA((NB,)),
            pltpu.SemaphoreType.DMA((3,)),
            pltpu.SemaphoreType.DMA((3,)),
            pltpu.SemaphoreType.DMA((3,)),
            pltpu.SemaphoreType.DMA((3,)),
        ],
        compiler_params=pltpu.CompilerParams(collective_id=0),
    )(x, Win0, Wout0, Win1, Wout1, Win2, Wout2)


# device time: 72800 ns/iter; 2.7422x vs baseline; 1.0388x over previous
import os

import jax
import jax.numpy as jnp
from jax import lax
from jax.experimental import pallas as pl
from jax.experimental.pallas import tpu as pltpu

N_DEV = 4
_NOCOMM = os.environ.get("KERNEL_NOCOMM") == "1"
_NODOT = os.environ.get("KERNEL_NODOT") == "1"


def kernel(x, Win0, Wout0, Win1, Wout1, Win2, Wout2):
    m, kdim = x.shape
    ndim = Win0.shape[1]
    odim = Wout0.shape[1]
    Q = ndim // N_DEV
    C = 512
    HC = Q // C
    NCH = ndim // C
    NB = 4

    def body(x_ref, win0, wout0, win1, wout1, win2, wout2, o_ref,
             act_bf, pown, p1send, p1recv, hbf, p2recv, acc,
             winbuf, woutbuf,
             win_sems, wout_sems,
             p1_send_sems, p1_recv_sems, p2_send_sems, p2_recv_sems):
        my = lax.axis_index("i")
        wins = (win0, win1, win2)
        wouts = (wout0, wout1, wout2)

        barrier = pltpu.get_barrier_semaphore()
        for d in range(1, N_DEV):
            pl.semaphore_signal(
                barrier, inc=1,
                device_id=(lax.rem(my + d, N_DEV),),
                device_id_type=pl.DeviceIdType.MESH,
            )
        pl.semaphore_wait(barrier, N_DEV - 1)

        def issue_win(L, t):
            slot = t % NB
            q = lax.rem(my + 1 + (t // HC), N_DEV)
            col = q * Q + (t % HC) * C
            cp = pltpu.make_async_copy(
                wins[L].at[:, pl.ds(col, C)], winbuf.at[slot],
                win_sems.at[slot])
            cp.start()
            return cp

        def issue_wout(L, t):
            slot = t % NB
            q = lax.rem(my + (t // HC), N_DEV)
            row = q * Q + (t % HC) * C
            cp = pltpu.make_async_copy(
                wouts[L].at[pl.ds(row, C), :], woutbuf.at[slot],
                wout_sems.at[slot])
            cp.start()
            return cp

        def p1_rdma(s):
            peer = lax.rem(my + 1 + s, N_DEV)
            return pltpu.make_async_remote_copy(
                src_ref=p1send.at[s], dst_ref=p1recv.at[2 - s],
                send_sem=p1_send_sems.at[s], recv_sem=p1_recv_sems.at[2 - s],
                device_id=(peer,), device_id_type=pl.DeviceIdType.MESH)

        def p2_rdma(s):
            peer = lax.rem(my + 1 + s, N_DEV)
            return pltpu.make_async_remote_copy(
                src_ref=hbf, dst_ref=p2recv.at[2 - s],
                send_sem=p2_send_sems.at[s], recv_sem=p2_recv_sems.at[2 - s],
                device_id=(peer,), device_id_type=pl.DeviceIdType.MESH)

        def p1_wait(slot):
            pltpu.make_async_remote_copy(
                src_ref=p1send.at[0], dst_ref=p1recv.at[slot],
                send_sem=p1_send_sems.at[0], recv_sem=p1_recv_sems.at[slot],
                device_id=(my,), device_id_type=pl.DeviceIdType.MESH,
            ).wait_recv()

        def p2_wait(slot):
            pltpu.make_async_remote_copy(
                src_ref=hbf, dst_ref=p2recv.at[slot],
                send_sem=p2_send_sems.at[0], recv_sem=p2_recv_sems.at[slot],
                device_id=(my,), device_id_type=pl.DeviceIdType.MESH,
            ).wait_recv()

        act_bf[...] = x_ref[...].astype(jnp.bfloat16)

        win_d = {}
        wout_d = {}
        for t in range(min(NB, NCH)):
            win_d[(0, t)] = issue_win(0, t)

        for L in range(3):
            p1_sent = []
            for t in range(NCH):
                win_d.pop((L, t)).wait()
                j, half = divmod(t, HC)
                if _NODOT:
                    part = winbuf[t % NB, :m, :] + 1.0
                else:
                    part = jnp.dot(
                        act_bf[...], winbuf[t % NB].astype(jnp.bfloat16),
                        preferred_element_type=jnp.float32)
                if j < 3:
                    p1send[j, :, half * C:(half + 1) * C] = (
                        part.astype(jnp.bfloat16))
                    if half == HC - 1 and not _NOCOMM:
                        r = p1_rdma(j)
                        r.start()
                        p1_sent.append(r)
                else:
                    pown[:, half * C:(half + 1) * C] = part
                if t + NB < NCH:
                    win_d[(L, t + NB)] = issue_win(L, t + NB)

            for t in range(min(NB, NCH)):
                wout_d[(L, t)] = issue_wout(L, t)

            if not _NOCOMM:
                for s in range(3):
                    p1_wait(s)
            h = (pown[...]
                 + p1recv[0].astype(jnp.float32)
                 + p1recv[1].astype(jnp.float32)
                 + p1recv[2].astype(jnp.float32))
            hbf[...] = jnp.maximum(h, 0.0).astype(jnp.bfloat16)
            p2_sent = []
            if not _NOCOMM:
                for s in range(3):
                    r = p2_rdma(s)
                    r.start()
                    p2_sent.append(r)

            for t in range(NCH):
                j, half = divmod(t, HC)
                if half == 0 and j > 0 and not _NOCOMM:
                    p2_wait(j - 1)
                hpart = (hbf[:, half * C:(half + 1) * C] if j == 0
                         else p2recv[j - 1, :, half * C:(half + 1) * C])
                wout_d.pop((L, t)).wait()
                if _NODOT:
                    term = woutbuf[t % NB, :m, :] + 1.0
                else:
                    term = jnp.dot(
                        hpart, woutbuf[t % NB].astype(jnp.bfloat16),
                        preferred_element_type=jnp.float32)
                if t == 0:
                    acc[...] = term
                else:
                    acc[...] += term
                if t + NB < NCH:
                    wout_d[(L, t + NB)] = issue_wout(L, t + NB)
                if L < 2 and t < min(NB, NCH):
                    win_d[(L + 1, t)] = issue_win(L + 1, t)

            for r in p1_sent:
                r.wait_send()
            for r in p2_sent:
                r.wait_send()

            if L < 2:
                act_bf[...] = acc[...].astype(jnp.bfloat16)
            else:
                o_ref[...] = acc[...]

    return pl.pallas_call(
        body,
        out_shape=jax.ShapeDtypeStruct((m, odim), jnp.float32),
        in_specs=[
            pl.BlockSpec(memory_space=pltpu.VMEM),
            pl.BlockSpec(memory_space=pl.ANY),
            pl.BlockSpec(memory_space=pl.ANY),
            pl.BlockSpec(memory_space=pl.ANY),
            pl.BlockSpec(memory_space=pl.ANY),
            pl.BlockSpec(memory_space=pl.ANY),
            pl.BlockSpec(memory_space=pl.ANY),
        ],
        out_specs=pl.BlockSpec(memory_space=pltpu.VMEM),
        scratch_shapes=[
            pltpu.VMEM((m, kdim), jnp.bfloat16),
            pltpu.VMEM((m, Q), jnp.float32),
            pltpu.VMEM((3, m, Q), jnp.bfloat16),
            pltpu.VMEM((3, m, Q), jnp.bfloat16),
            pltpu.VMEM((m, Q), jnp.bfloat16),
            pltpu.VMEM((3, m, Q), jnp.bfloat16),
            pltpu.VMEM((m, odim), jnp.float32),
            pltpu.VMEM((NB, kdim, C), jnp.float32),
            pltpu.VMEM((NB, C, odim), jnp.float32),
            pltpu.SemaphoreType.DMA((NB,)),
            pltpu.SemaphoreType.DMA((NB,)),
            pltpu.SemaphoreType.DMA((3,)),
            pltpu.SemaphoreType.DMA((3,)),
            pltpu.SemaphoreType.DMA((3,)),
            pltpu.SemaphoreType.DMA((3,)),
        ],
        compiler_params=pltpu.CompilerParams(
            collective_id=0, vmem_limit_bytes=60 * 1024 * 1024),
    )(x, Win0, Wout0, Win1, Wout1, Win2, Wout2)
